# Initial kernel scaffold; baseline (speedup 1.0000x reference)
import jax
import jax.numpy as jnp
from jax.experimental import pallas as pl


def kernel(x, edge_index, W_i1, b_i1, W_i2, b_i2, W_p1, b_p1, W_p2, b_p2, W_l, b_l, W_g, b_g, W_f, b_f, W_m1, b_m1, W_m2, b_m2, W_m3, b_m3, P):
    raise NotImplementedError("write your pallas kernel here")


# speedup vs baseline: 3.2017x; 3.2017x over previous
import functools

import jax
import jax.numpy as jnp
from jax import lax
from jax.experimental import pallas as pl
from jax.experimental.pallas import tpu as pltpu
from jax.experimental.pallas import tpu_sc as plsc

NC, NS = 2, 16
NW = NC * NS
LANE = 16

N, E, D, H, C = 10000, 320000, 128, 128, 8
EPAD = 327680
NR = 320
NPAD = NW * NR
BN = 1000
CH = 1280
NCHUNK = E // CH
STAGE_CAP = 288
FLUSH_AT = 272
EPW = E // NW
GGRP = 1000
CAPR = E + 512
RB = 128

_mesh = functools.partial(
    plsc.VectorSubcoreMesh, core_axis_name="c", subcore_axis_name="s",
    num_cores=NC, num_subcores=NS)


def _wid():
    return lax.axis_index("s") * NC + lax.axis_index("c")


def _logits_body(x_ref, wm1, bm1, wm2, bm2, wm3, bm3, p_ref, a_ref, b_ref):
    xb = x_ref[...]
    h = jax.nn.relu(jnp.dot(xb, wm1[...], preferred_element_type=jnp.float32)
                    + bm1[...])
    h = jax.nn.relu(jnp.dot(h, wm2[...], preferred_element_type=jnp.float32)
                    + bm2[...])
    lg = jnp.dot(h, wm3[...], preferred_element_type=jnp.float32) + bm3[...]
    parsing = jax.nn.relu(2.0 * p_ref[...])
    bb = jnp.dot(lg, parsing, preferred_element_type=jnp.float32)
    pad = jnp.zeros((lg.shape[0], 16 - C), jnp.float32)
    a_ref[...] = jnp.concatenate([lg, pad], axis=1)
    b_ref[...] = jnp.concatenate([bb, pad], axis=1)


def _uc_body(x_ref, wi1, bi1, wi2, bi2, wp1, bp1, wp2, bp2, wla, wlb, bl,
             u_ref, c_ref):
    xb = x_ref[...]
    h0 = jax.nn.relu(jnp.dot(xb, wi1[...], preferred_element_type=jnp.float32)
                     + bi1[...])
    h0 = jnp.dot(h0, wi2[...], preferred_element_type=jnp.float32) + bi2[...]
    p = jax.nn.relu(jnp.dot(h0, wp1[...], preferred_element_type=jnp.float32)
                    + bp1[...])
    pos = jnp.dot(p, wp2[...], preferred_element_type=jnp.float32) + bp2[...]
    pb = jnp.dot(pos, wlb[...], preferred_element_type=jnp.float32)
    u_ref[...] = jnp.dot(h0, wla[...], preferred_element_type=jnp.float32) + pb
    c_ref[...] = bl[...] - pb


def _ew_body(ae_ref, be_ref, ew_ref, st_ref, acc):
    i = pl.program_id(0)

    @pl.when(i == 0)
    def _():
        acc[0] = 0.0
        acc[1] = 0.0

    br = ae_ref.shape[0]
    prod = ae_ref[...] * be_ref[...]
    jj = lax.broadcasted_iota(jnp.int32, (2048, 128), 0)
    kk = lax.broadcasted_iota(jnp.int32, (2048, 128), 1)
    sel = ((jj // 16) == kk).astype(jnp.float32)
    rs = jax.lax.dot_general(prod, sel, (((1,), (0,)), ((), ())),
                             precision=jax.lax.Precision.HIGHEST,
                             preferred_element_type=jnp.float32)
    grow = lax.broadcasted_iota(jnp.int32, rs.shape, 0) + i * br
    rs = jnp.where(grow < (E * 16) // 2048, rs, 0.0)
    ew_ref[...] = rs
    acc[0] += jnp.sum(rs)
    acc[1] += jnp.sum(rs * rs)
    st_ref[0] = acc[0]
    st_ref[1] = acc[1]


def _between_body(agg_ref, c_ref, wg, bg, wla, bl, u_ref):
    xh = jax.nn.relu(jnp.dot(agg_ref[...], wg[...],
                             preferred_element_type=jnp.float32) + bg[...])
    u_ref[...] = (jnp.dot(xh, wla[...], preferred_element_type=jnp.float32)
                  + bl[...] - c_ref[...])


def _final_body(agg_ref, wg, bg, wf, bf, o_ref):
    xh = jax.nn.relu(jnp.dot(agg_ref[...], wg[...],
                             preferred_element_type=jnp.float32) + bg[...])
    o_ref[...] = jnp.dot(xh, wf[...], preferred_element_type=jnp.float32) + bf[...]


def _full(shape):
    return pl.BlockSpec(shape, lambda i: (0,) * len(shape))


def _rows(bn, k):
    return pl.BlockSpec((bn, k), lambda i: (i, 0))


def _gather_ab_body(a_hbm, b_hbm, src_hbm, dst_hbm, ae_hbm, be_hbm,
                    sidx, didx, abuf, bbuf, sem):
    wid = _wid()
    e0 = wid * EPW
    pltpu.sync_copy(src_hbm.at[pl.ds(e0, EPW)], sidx)
    pltpu.sync_copy(dst_hbm.at[pl.ds(e0, EPW)], didx)
    pieces = [(j * 128, 128) for j in range(7)] + [(896, 104)]

    @pl.loop(0, EPW // GGRP)
    def _(g):
        roff = g * GGRP
        cps = []
        for off, sz in pieces:
            cp = pltpu.make_async_copy(
                a_hbm.at[sidx.at[pl.ds(roff + off, sz)]],
                abuf.at[pl.ds(off, sz)], sem)
            cp.start()
            cps.append(cp)
            cp = pltpu.make_async_copy(
                b_hbm.at[didx.at[pl.ds(roff + off, sz)]],
                bbuf.at[pl.ds(off, sz)], sem)
            cp.start()
            cps.append(cp)
        for cp in cps:
            cp.wait()
        pltpu.sync_copy(abuf, ae_hbm.at[pl.ds(e0 + roff, GGRP)])
        pltpu.sync_copy(bbuf, be_hbm.at[pl.ds(e0 + roff, GGRP)])


def _init_acc_cloc(acc, c_loc):
    neg_inf = jnp.full((LANE,), -jnp.inf, jnp.float32)
    zero = jnp.zeros((LANE,), jnp.float32)

    @pl.loop(0, NR + 8)
    def _(r):
        for j in range(8):
            acc[r, pl.ds(j * LANE, LANE)] = neg_inf

    for j in range(8):
        c_loc[NR, pl.ds(j * LANE, LANE)] = zero


def _finish_acc(acc, out_hbm, base):
    zero = jnp.zeros((LANE,), jnp.float32)

    @pl.loop(0, NR)
    def _(r):
        for j in range(8):
            cs = pl.ds(j * LANE, LANE)
            v = acc[r, cs]
            acc[r, cs] = jnp.where(v == -jnp.inf, zero, v)

    pltpu.sync_copy(acc.at[pl.ds(0, NR)], out_hbm.at[pl.ds(base, NR)])


def _layer_body(u_hbm, c_hbm, src_hbm, dst_hbm, ew_hbm, al_hbm, be_hbm,
                out_hbm, rsrc_hbm, rdl_hbm, rw_hbm, cnt_hbm,
                c_loc, acc, u_buf, st_src, st_dl, st_ew,
                dbuf, sbuf, wbuf, av, bv, cbuf, cnt_ref, wcnt,
                sem0, sem1, gsem, dsem):
    wid = _wid()
    base = wid * NR
    pltpu.sync_copy(c_hbm.at[pl.ds(base, NR)], c_loc.at[pl.ds(0, NR)])
    pltpu.sync_copy(al_hbm, av)
    pltpu.sync_copy(be_hbm, bv)
    alpha_v = av[...]
    beta_v = bv[...]

    _init_acc_cloc(acc, c_loc)
    cnt_ref[0] = 0
    wcnt[0] = 0

    def start_chunk(g, buf_slot, sem):
        pltpu.make_async_copy(dst_hbm.at[pl.ds(g * CH, CH)],
                              dbuf.at[buf_slot], sem).start()
        pltpu.make_async_copy(src_hbm.at[pl.ds(g * CH, CH)],
                              sbuf.at[buf_slot], sem).start()
        pltpu.make_async_copy(ew_hbm.at[pl.ds(g * CH, CH)],
                              wbuf.at[buf_slot], sem).start()

    def wait_chunk(buf_slot, sem):
        pltpu.make_async_copy(dst_hbm.at[pl.ds(0, CH)],
                              dbuf.at[buf_slot], sem).wait()
        pltpu.make_async_copy(src_hbm.at[pl.ds(0, CH)],
                              sbuf.at[buf_slot], sem).wait()
        pltpu.make_async_copy(ew_hbm.at[pl.ds(0, CH)],
                              wbuf.at[buf_slot], sem).wait()

    def flush():
        woff = pl.multiple_of(wcnt[0], 256)
        dumps = [
            pltpu.make_async_copy(st_src.at[pl.ds(0, 256)],
                                  rsrc_hbm.at[pl.ds(pl.multiple_of(wid * CAPR + woff, 256), 256)],
                                  dsem),
            pltpu.make_async_copy(st_dl.at[pl.ds(0, 256)],
                                  rdl_hbm.at[pl.ds(pl.multiple_of(wid * CAPR + woff, 256), 256)],
                                  dsem),
            pltpu.make_async_copy(st_ew.at[pl.ds(0, 256)],
                                  rw_hbm.at[pl.ds(pl.multiple_of(wid * CAPR + woff, 256), 256)],
                                  dsem),
        ]
        for cp in dumps:
            cp.start()
        cp0 = pltpu.make_async_copy(u_hbm.at[st_src.at[pl.ds(0, 128)]],
                                    u_buf.at[pl.ds(0, 128)], gsem)
        cp1 = pltpu.make_async_copy(u_hbm.at[st_src.at[pl.ds(128, 128)]],
                                    u_buf.at[pl.ds(128, 128)], gsem)
        cp0.start()
        cp1.start()
        cp0.wait()
        cp1.wait()

        @pl.loop(0, 256 // LANE)
        def _(v):
            o16 = v * LANE
            dlv = st_dl[pl.ds(o16, LANE)]
            wv = st_ew[pl.ds(o16, LANE)]
            for k in range(LANE):
                dl = dlv[k]
                w = wv[k]
                i = o16 + k
                for j in range(8):
                    cs = pl.ds(j * LANE, LANE)
                    m = w * (u_buf[i, cs] + c_loc[dl, cs])
                    acc[dl, cs] = jnp.maximum(acc[dl, cs], m)

        for cp in dumps:
            cp.wait()
        for stg in (st_src, st_dl, st_ew):
            t0 = stg[pl.ds(256, LANE)]
            t1 = stg[pl.ds(256 + LANE, LANE)]
            stg[pl.ds(0, LANE)] = t0
            stg[pl.ds(LANE, LANE)] = t1
        cnt_ref[0] = cnt_ref[0] - 256
        wcnt[0] = woff + 256

    def process(g, buf_slot):
        @pl.loop(0, CH // LANE)
        def _(v):
            off = v * LANE
            d = dbuf[buf_slot, pl.ds(off, LANE)]
            s_ = sbuf[buf_slot, pl.ds(off, LANE)]
            w = wbuf[buf_slot, pl.ds(off, LANE)]
            dl = d - base
            msk = (dl >= 0) & (dl < NR)
            wn = alpha_v * w + beta_v
            cnt = cnt_ref[0]
            plsc.store_compressed(st_src.at[pl.ds(cnt, LANE)], s_, mask=msk)
            plsc.store_compressed(st_dl.at[pl.ds(cnt, LANE)], dl, mask=msk)
            plsc.store_compressed(st_ew.at[pl.ds(cnt, LANE)], wn, mask=msk)
            nadd = jnp.sum(jnp.where(msk, 1, 0))
            cnt2 = cnt + nadd
            cnt_ref[0] = cnt2

            @pl.when(cnt2 >= FLUSH_AT)
            def _():
                flush()

    start_chunk(0, 0, sem0)

    @pl.loop(0, NCHUNK, step=2)
    def _(g):
        @pl.when(g + 1 < NCHUNK)
        def _():
            start_chunk(g + 1, 1, sem1)

        wait_chunk(0, sem0)
        process(g, 0)

        @pl.when(g + 2 < NCHUNK)
        def _():
            start_chunk(g + 2, 0, sem0)

        @pl.when(g + 1 < NCHUNK)
        def _():
            wait_chunk(1, sem1)
            process(g + 1, 1)

    @pl.when(cnt_ref[0] >= 256)
    def _():
        flush()

    @pl.when(cnt_ref[0] > 0)
    def _():
        rem = cnt_ref[0]
        tr_s = jnp.zeros((LANE,), jnp.int32)
        tr_d = jnp.full((LANE,), NR, jnp.int32)
        tr_w = jnp.zeros((LANE,), jnp.float32)
        for t in range(16):
            ct = rem + t * LANE

            @pl.when(ct < 256)
            def _():
                st_src[pl.ds(ct, LANE)] = tr_s
                st_dl[pl.ds(ct, LANE)] = tr_d
                st_ew[pl.ds(ct, LANE)] = tr_w

        flush()

    cbuf[...] = jnp.full((LANE,), 0, jnp.int32) + wcnt[0]
    pltpu.sync_copy(cbuf, cnt_hbm.at[pl.ds(pl.multiple_of(wid * LANE, LANE), LANE)])
    _finish_acc(acc, out_hbm, base)


def _replay_body(u_hbm, c_hbm, rsrc_hbm, rdl_hbm, rw_hbm, cnt_hbm, out_hbm,
                 c_loc, acc, u_buf, rs_b, rd_b, rw_b, cbuf,
                 rsem0, rsem1, gsem0, gsem1):
    wid = _wid()
    base = wid * NR
    pltpu.sync_copy(c_hbm.at[pl.ds(base, NR)], c_loc.at[pl.ds(0, NR)])
    pltpu.sync_copy(cnt_hbm.at[pl.ds(pl.multiple_of(wid * LANE, LANE), LANE)], cbuf)
    cnt = cbuf[...][0]
    nblk = cnt // RB
    _init_acc_cloc(acc, c_loc)

    def r_start(b, slot, sem):
        off = pl.multiple_of(wid * CAPR + b * RB, RB)
        pltpu.make_async_copy(rsrc_hbm.at[pl.ds(off, RB)],
                              rs_b.at[slot], sem).start()
        pltpu.make_async_copy(rdl_hbm.at[pl.ds(off, RB)],
                              rd_b.at[slot], sem).start()
        pltpu.make_async_copy(rw_hbm.at[pl.ds(off, RB)],
                              rw_b.at[slot], sem).start()

    def r_wait(slot, sem):
        pltpu.make_async_copy(rsrc_hbm.at[pl.ds(0, RB)],
                              rs_b.at[slot], sem).wait()
        pltpu.make_async_copy(rdl_hbm.at[pl.ds(0, RB)],
                              rd_b.at[slot], sem).wait()
        pltpu.make_async_copy(rw_hbm.at[pl.ds(0, RB)],
                              rw_b.at[slot], sem).wait()

    def g_start(slot, sem):
        pltpu.make_async_copy(u_hbm.at[rs_b.at[slot]],
                              u_buf.at[slot], sem).start()

    def g_wait(slot, sem):
        pltpu.make_async_copy(u_hbm.at[rs_b.at[slot]],
                              u_buf.at[slot], sem).wait()

    def process(slot):
        @pl.loop(0, RB // LANE)
        def _(v):
            o16 = v * LANE
            dlv = rd_b[slot, pl.ds(o16, LANE)]
            wv = rw_b[slot, pl.ds(o16, LANE)]
            for k in range(LANE):
                dl = dlv[k]
                w = wv[k]
                i = o16 + k
                for j in range(8):
                    cs = pl.ds(j * LANE, LANE)
                    m = w * (u_buf[slot, i, cs] + c_loc[dl, cs])
                    acc[dl, cs] = jnp.maximum(acc[dl, cs], m)

    @pl.when(nblk > 0)
    def _():
        r_start(0, 0, rsem0)
        r_wait(0, rsem0)
        g_start(0, gsem0)

        @pl.when(nblk > 1)
        def _():
            r_start(1, 1, rsem1)

    def pair(p, carry):
        b0 = 2 * p
        b1 = b0 + 1
        g_wait(0, gsem0)

        @pl.when(b1 < nblk)
        def _():
            r_wait(1, rsem1)
            g_start(1, gsem1)

        process(0)

        @pl.when(b0 + 2 < nblk)
        def _():
            r_start(b0 + 2, 0, rsem0)

        @pl.when(b1 < nblk)
        def _():
            g_wait(1, gsem1)

        @pl.when(b0 + 2 < nblk)
        def _():
            r_wait(0, rsem0)
            g_start(0, gsem0)

        @pl.when(b1 < nblk)
        def _():
            process(1)

        @pl.when(b1 + 2 < nblk)
        def _():
            r_start(b1 + 2, 1, rsem1)

        return carry

    lax.fori_loop(0, (nblk + 1) // 2, pair, 0)
    _finish_acc(acc, out_hbm, base)


def _scan_layer(u, c_pad, src, dst, ew_raw, alpha16, beta16):
    f = pl.kernel(
        _layer_body,
        out_type=(jax.ShapeDtypeStruct((NPAD, H), jnp.float32),
                  jax.ShapeDtypeStruct((NW * CAPR,), jnp.int32),
                  jax.ShapeDtypeStruct((NW * CAPR,), jnp.int32),
                  jax.ShapeDtypeStruct((NW * CAPR,), jnp.float32),
                  jax.ShapeDtypeStruct((NW * LANE,), jnp.int32)),
        mesh=_mesh(),
        scratch_types=[
            pltpu.VMEM((NR + 8, H), jnp.float32),
            pltpu.VMEM((NR + 8, H), jnp.float32),
            pltpu.VMEM((256, H), jnp.float32),
            pltpu.VMEM((STAGE_CAP,), jnp.int32),
            pltpu.VMEM((STAGE_CAP,), jnp.int32),
            pltpu.VMEM((STAGE_CAP,), jnp.float32),
            pltpu.VMEM((2, CH), jnp.int32),
            pltpu.VMEM((2, CH), jnp.int32),
            pltpu.VMEM((2, CH), jnp.float32),
            pltpu.VMEM((LANE,), jnp.float32),
            pltpu.VMEM((LANE,), jnp.float32),
            pltpu.VMEM((LANE,), jnp.int32),
            pltpu.SMEM((1,), jnp.int32),
            pltpu.SMEM((1,), jnp.int32),
            pltpu.SemaphoreType.DMA,
            pltpu.SemaphoreType.DMA,
            pltpu.SemaphoreType.DMA,
            pltpu.SemaphoreType.DMA,
        ],
        compiler_params=pltpu.CompilerParams(needs_layout_passes=False),
    )
    return f(u, c_pad, src, dst, ew_raw, alpha16, beta16)


def _replay_layer(u, c_pad, rsrc, rdl, rw, cnts):
    f = pl.kernel(
        _replay_body,
        out_type=jax.ShapeDtypeStruct((NPAD, H), jnp.float32),
        mesh=_mesh(),
        scratch_types=[
            pltpu.VMEM((NR + 8, H), jnp.float32),
            pltpu.VMEM((NR + 8, H), jnp.float32),
            pltpu.VMEM((2, RB, H), jnp.float32),
            pltpu.VMEM((2, RB), jnp.int32),
            pltpu.VMEM((2, RB), jnp.int32),
            pltpu.VMEM((2, RB), jnp.float32),
            pltpu.VMEM((LANE,), jnp.int32),
            pltpu.SemaphoreType.DMA,
            pltpu.SemaphoreType.DMA,
            pltpu.SemaphoreType.DMA,
            pltpu.SemaphoreType.DMA,
        ],
        compiler_params=pltpu.CompilerParams(needs_layout_passes=False),
    )
    return f(u, c_pad, rsrc, rdl, rw, cnts)


def kernel(x, edge_index, W_i1, b_i1, W_i2, b_i2, W_p1, b_p1, W_p2, b_p2,
           W_l, b_l, W_g, b_g, W_f, b_f, W_m1, b_m1, W_m2, b_m2, W_m3, b_m3,
           P):
    src = edge_index[0].astype(jnp.int32)
    dst = edge_index[1].astype(jnp.int32)
    nblk = N // BN

    a_p, b_p = pl.pallas_call(
        _logits_body,
        grid=(nblk,),
        in_specs=[_rows(BN, D), _full((D, 512)), _full((1, 512)),
                  _full((512, 64)), _full((1, 64)), _full((64, C)),
                  _full((1, C)), _full((C, C))],
        out_specs=[_rows(BN, 16), _rows(BN, 16)],
        out_shape=[jax.ShapeDtypeStruct((N, 16), jnp.float32),
                   jax.ShapeDtypeStruct((N, 16), jnp.float32)],
    )(x, W_m1, b_m1.reshape(1, -1), W_m2, b_m2.reshape(1, -1),
      W_m3, b_m3.reshape(1, -1), P)

    ae, be = pl.kernel(
        _gather_ab_body,
        out_type=(jax.ShapeDtypeStruct((EPAD, 16), jnp.float32),
                  jax.ShapeDtypeStruct((EPAD, 16), jnp.float32)),
        mesh=_mesh(),
        scratch_types=[
            pltpu.VMEM((EPW,), jnp.int32),
            pltpu.VMEM((EPW,), jnp.int32),
            pltpu.VMEM((GGRP, 16), jnp.float32),
            pltpu.VMEM((GGRP, 16), jnp.float32),
            pltpu.SemaphoreType.DMA,
        ],
        compiler_params=pltpu.CompilerParams(use_tc_tiling_on_sc=False),
    )(a_p, b_p, src, dst)

    wla = W_l[:H]
    wlb = W_l[H:]
    u1, c = pl.pallas_call(
        _uc_body,
        grid=(nblk,),
        in_specs=[_rows(BN, D), _full((D, H)), _full((1, H)),
                  _full((H, H)), _full((1, H)), _full((H, H)), _full((1, H)),
                  _full((H, 3)), _full((1, 3)), _full((H, H)), _full((3, H)),
                  _full((1, H))],
        out_specs=[_rows(BN, H), _rows(BN, H)],
        out_shape=[jax.ShapeDtypeStruct((N, H), jnp.float32),
                   jax.ShapeDtypeStruct((N, H), jnp.float32)],
    )(x, W_i1, b_i1.reshape(1, -1), W_i2, b_i2.reshape(1, -1),
      W_p1, b_p1.reshape(1, -1), W_p2, b_p2.reshape(1, -1), wla, wlb,
      b_l.reshape(1, -1))

    BR = 512
    ae2 = ae.reshape(2560, 2048)
    be2 = be.reshape(2560, 2048)
    ew2, stats = pl.pallas_call(
        _ew_body,
        grid=(2560 // BR,),
        in_specs=[pl.BlockSpec((BR, 2048), lambda i: (i, 0)),
                  pl.BlockSpec((BR, 2048), lambda i: (i, 0))],
        out_specs=[pl.BlockSpec((BR, 128), lambda i: (i, 0)),
                   pl.BlockSpec(memory_space=pltpu.SMEM)],
        out_shape=[jax.ShapeDtypeStruct((2560, 128), jnp.float32),
                   jax.ShapeDtypeStruct((2,), jnp.float32)],
        scratch_shapes=[pltpu.SMEM((2,), jnp.float32)],
    )(ae2, be2)
    ew_raw = ew2.reshape(EPAD)

    s = stats[0]
    ss = stats[1]
    mean = s / E
    var = (ss - s * s / E) / (E - 1)
    alpha = jnp.sqrt(1e-4 / var)
    beta = 1.0 - mean * alpha
    alpha16 = jnp.full((LANE,), alpha, jnp.float32)
    beta16 = jnp.full((LANE,), beta, jnp.float32)

    c_pad = jnp.pad(c, ((0, NPAD - N), (0, 0)))

    agg1p, rsrc, rdl, rw, cnts = _scan_layer(u1, c_pad, src, dst, ew_raw,
                                             alpha16, beta16)
    agg1 = agg1p[:N]

    u2 = pl.pallas_call(
        _between_body,
        grid=(nblk,),
        in_specs=[_rows(BN, H), _rows(BN, H), _full((H, H)), _full((1, H)),
                  _full((H, H)), _full((1, H))],
        out_specs=_rows(BN, H),
        out_shape=jax.ShapeDtypeStruct((N, H), jnp.float32),
    )(agg1, c, W_g, b_g.reshape(1, -1), wla, b_l.reshape(1, -1))

    agg2 = _replay_layer(u2, c_pad, rsrc, rdl, rw, cnts)[:N]

    out = pl.pallas_call(
        _final_body,
        grid=(nblk,),
        in_specs=[_rows(BN, H), _full((H, H)), _full((1, H)),
                  _full((H, C)), _full((1, C))],
        out_specs=_rows(BN, C),
        out_shape=jax.ShapeDtypeStruct((N, C), jnp.float32),
    )(agg2, W_g, b_g.reshape(1, -1), W_f, b_f.reshape(1, -1))

    return out
